# baseline (speedup 1.0000x reference)
import functools

import jax
import jax.numpy as jnp
from jax import lax
from jax.experimental import pallas as pl
from jax.experimental.pallas import tpu as pltpu
from jax.experimental.pallas import tpu_sc as plsc

NUM_CORES = 2
NUM_SUBCORES = 16
NUM_WORKERS = NUM_CORES * NUM_SUBCORES


def _make_sc_gather(B, V, D, b_per_w, chunk, nbuf):
  nchunks = b_per_w // chunk
  assert b_per_w % chunk == 0 and chunk <= 128 and nchunks >= nbuf
  mesh = plsc.VectorSubcoreMesh(
      core_axis_name="c", subcore_axis_name="s",
      num_cores=NUM_CORES, num_subcores=NUM_SUBCORES)

  @functools.partial(
      pl.kernel,
      out_type=jax.ShapeDtypeStruct((B, D), jnp.float32),
      mesh=mesh,
      scratch_types=[
          pltpu.VMEM((b_per_w,), jnp.int32),
          pltpu.VMEM_SHARED((NUM_SUBCORES * 16, D), jnp.float32),
      ] + [pltpu.VMEM((chunk, D), jnp.float32)] * nbuf
        + [pltpu.SemaphoreType.DMA] * (2 * nbuf),
      compiler_params=pltpu.CompilerParams(
          use_tc_tiling_on_sc=False, needs_layout_passes=False),
  )
  def sc_gather(ids_hbm, table_hbm, out_hbm, idx_v, table_sh, *bufs_and_sems):
    bufs = bufs_and_sems[:nbuf]
    gsems = bufs_and_sems[nbuf:2 * nbuf]
    wsems = bufs_and_sems[2 * nbuf:]
    sid = lax.axis_index("s")
    wid = sid * NUM_CORES + lax.axis_index("c")
    base = pl.multiple_of(wid * b_per_w, b_per_w)
    tcopy = pltpu.make_async_copy(
        table_hbm, table_sh.at[pl.ds(pl.multiple_of(sid * 16, 16), V)],
        gsems[0])
    tcopy.start()
    pltpu.sync_copy(ids_hbm.at[pl.ds(base, b_per_w)], idx_v)
    soff = sid * 16
    for i in range(b_per_w // 16):
      sl = pl.ds(i * 16, 16)
      idx_v[sl] = idx_v[sl] + soff
    tcopy.wait()

    def gather(c):
      b = c % nbuf
      off = pl.multiple_of(c * chunk, chunk)
      return pltpu.make_async_copy(
          table_sh.at[idx_v.at[pl.ds(off, chunk)]], bufs[b], gsems[b])

    def writeout(c):
      b = c % nbuf
      off = pl.multiple_of(c * chunk, chunk)
      return pltpu.make_async_copy(
          bufs[b], out_hbm.at[pl.ds(base + off, chunk)], wsems[b])

    for c in range(nbuf - 1):
      gather(c).start()
    for c in range(nchunks):
      if c + nbuf - 1 < nchunks:
        if c >= 1:
          writeout(c - 1).wait()
        gather(c + nbuf - 1).start()
      gather(c).wait()
      writeout(c).start()
    for c in range(max(nchunks - nbuf + 1, 1), nchunks):
      writeout(c - 1).wait()
    writeout(nchunks - 1).wait()

  return sc_gather


def kernel(word_ids, table):
  B = word_ids.shape[0]
  V, D = table.shape
  b_per_w = B // NUM_WORKERS
  ids = word_ids[:, 0].astype(jnp.int32)
  f = _make_sc_gather(B, V, D, b_per_w, 32, 4)
  return f(ids, table)
